# baseline (speedup 1.0000x reference)
import functools

import numpy as np
import jax
import jax.numpy as jnp
from jax import lax
from jax.experimental import pallas as pl
from jax.experimental.pallas import tpu as pltpu
from jax.experimental.pallas import tpu_sc as plsc

_B = 4
_N = 4096
_K = 16
_L = 16
_NBLK = _N // _L
_NC = 2
_NS = 16
_NW = _NC * _NS
_WPB = _NW // _B
_ROWS = _N // _WPB
_BIG = np.float32(3.0e38)
_SAMPLE = 256
_HCAP = _N + _L


def _sqrt16(a):
    i = plsc.bitcast(a, jnp.int32)
    i = jnp.int32(0x5F3759DF) - (i >> 1)
    y = plsc.bitcast(i, jnp.float32)
    ah = a * jnp.float32(0.5)
    y = y * (jnp.float32(1.5) - ah * y * y)
    y = y * (jnp.float32(1.5) - ah * y * y)
    y = y * (jnp.float32(1.5) - ah * y * y)
    return jnp.where(a > 0.0, a * y, jnp.float32(0.0))


def _body(rx_hbm, ry_hbm, rz_hbm, px_hbm, py_hbm, pz_hbm, out_hbm,
          xs, ys, zs, pxs, pys, pzs, sqc, hitv, hitv2, hitv3, hitv4, accv):
    wid = lax.axis_index("s") * _NC + lax.axis_index("c")
    batch = wid // _WPB
    row0 = (wid % _WPB) * _ROWS

    boff = batch * _N
    pltpu.sync_copy(rx_hbm.at[pl.ds(boff, _N)], xs)
    pltpu.sync_copy(ry_hbm.at[pl.ds(boff, _N)], ys)
    pltpu.sync_copy(rz_hbm.at[pl.ds(boff, _N)], zs)
    pltpu.sync_copy(px_hbm.at[pl.ds(boff, _N)], pxs)
    pltpu.sync_copy(py_hbm.at[pl.ds(boff, _N)], pys)
    pltpu.sync_copy(pz_hbm.at[pl.ds(boff, _N)], pzs)

    iota = lax.iota(jnp.int32, _L)

    @plsc.parallel_loop(0, _NBLK, unroll=4)
    def _sq(c):
        base = c * _L
        xv = xs[pl.ds(base, _L)]
        yv = ys[pl.ds(base, _L)]
        zv = zs[pl.ds(base, _L)]
        sqc[pl.ds(base, _L)] = xv * xv + yv * yv + zv * zv

    def key_block(base, q):
        xv = xs[pl.ds(base, _L)]
        yv = ys[pl.ds(base, _L)]
        zv = zs[pl.ds(base, _L)]
        sc = sqc[pl.ds(base, _L)]
        t0 = q[0] * xv + q[1] * yv + q[2] * zv
        return sc - 2.0 * t0

    def merge(carry, d2, idxv):
        bk, bv = carry
        sk, sv = plsc.sort_key_val(d2, idxv, descending=True)
        take = bk <= sk
        mk = jnp.where(take, bk, sk)
        mv = jnp.where(take, bv, sv)
        nk, nv = plsc.sort_key_val(mk, mv)
        return nk, nv

    def sample_top16x4(qs, rvs):
        def p1_body(c, carry):
            base = c * _L
            xv = xs[pl.ds(base, _L)]
            yv = ys[pl.ds(base, _L)]
            zv = zs[pl.ds(base, _L)]
            sc = sqc[pl.ds(base, _L)]
            idxv = iota + base
            out = []
            for q, rv, ch in zip(qs, rvs, carry):
                v = sc - 2.0 * (q[0] * xv + q[1] * yv + q[2] * zv)
                v = jnp.where(idxv == rv, _BIG, v)
                out.append(merge(ch, v, idxv))
            return tuple(out)

        bk0 = jnp.full((_L,), _BIG, jnp.float32)
        bv0 = jnp.zeros((_L,), jnp.int32)
        init = tuple((bk0, bv0) for _ in range(4))
        return lax.fori_loop(0, _SAMPLE // _L, p1_body, init)

    def merge_hits_x4(chains, cnts, hrefs, rvs, qs):
        def p3_body(j, carry):
            base = j * _L
            out = []
            for (bk, bv), cnt, hv_ref, rv, q in zip(
                    carry, cnts, hrefs, rvs, qs):
                valid = iota + base < cnt
                hv = hv_ref[pl.ds(base, _L)]
                cx = plsc.load_gather(xs, [hv], mask=valid)
                cy = plsc.load_gather(ys, [hv], mask=valid)
                cz = plsc.load_gather(zs, [hv], mask=valid)
                scv = plsc.load_gather(sqc, [hv], mask=valid)
                hk = scv - 2.0 * (q[0] * cx + q[1] * cy + q[2] * cz)
                hk = jnp.where(valid, hk, _BIG)
                hk = jnp.where(hv == rv, _BIG, hk)
                out.append(merge((bk, bv), hk, hv))
            return tuple(out)

        cmax = jnp.maximum(jnp.maximum(cnts[0], cnts[1]),
                           jnp.maximum(cnts[2], cnts[3]))
        nit = (cmax + _L - 1) // _L
        return lax.fori_loop(0, nit, p3_body, chains)

    def edge_loss(rv, bk, bv):
        sqq = plsc.load_gather(sqc, [rv])
        dist_ref = _sqrt16(bk + sqq)
        qpx = plsc.load_gather(pxs, [rv])
        qpy = plsc.load_gather(pys, [rv])
        qpz = plsc.load_gather(pzs, [rv])
        nx = plsc.load_gather(pxs, [bv])
        ny = plsc.load_gather(pys, [bv])
        nz = plsc.load_gather(pzs, [bv])
        ddx = nx - qpx
        ddy = ny - qpy
        ddz = nz - qpz
        dist = _sqrt16(ddx * ddx + ddy * ddy + ddz * ddz)
        return jnp.abs(dist_ref - dist)

    def quad_body(p, acc_comp):
        acc, comp = acc_comp
        ra = row0 + 4 * p
        rva = jnp.full((_L,), ra, jnp.int32)
        rvs = (rva, rva + 1, rva + 2, rva + 3)
        qs = tuple(tuple(plsc.load_gather(s, [rv]) for s in (xs, ys, zs))
                   for rv in rvs)

        chains = sample_top16x4(qs, rvs)
        ts = tuple(jnp.max(ch[0]) for ch in chains)

        hrefs = (hitv, hitv2, hitv3, hitv4)
        zero4 = (jnp.zeros((_L,), jnp.int32),) * 4

        @plsc.parallel_loop(_SAMPLE // _L, _NBLK, unroll=8, carry=zero4)
        def p2_cnt(c, cnts):
            base = c * _L
            xv = xs[pl.ds(base, _L)]
            yv = ys[pl.ds(base, _L)]
            zv = zs[pl.ds(base, _L)]
            sc = sqc[pl.ds(base, _L)]
            idxv = iota + base
            out = []
            for q, t, hv_ref, cnt in zip(qs, ts, hrefs, cnts):
                v = sc - 2.0 * (q[0] * xv + q[1] * yv + q[2] * zv)
                hit = v < t
                hi = hit.astype(jnp.int32)
                pos = cnt + plsc.cumsum(hi) - hi
                plsc.store_scatter(hv_ref, [pos], idxv, mask=hit)
                out.append(cnt + plsc.all_reduce_population_count(hit))
            return tuple(out)

        cnts = tuple(cv[0] for cv in p2_cnt)
        chains = merge_hits_x4(chains, cnts, hrefs, rvs, qs)
        term = jnp.zeros((_L,), jnp.float32)
        for (bk, bv), rv in zip(chains, rvs):
            term = term + edge_loss(rv, bk, bv)

        y = term - comp
        t = acc + y
        comp = (t - acc) - y
        return t, comp

    zero = jnp.zeros((_L,), jnp.float32)
    acc, _ = lax.fori_loop(0, _ROWS // 4, quad_body, (zero, zero))
    accv[...] = acc
    pltpu.sync_copy(accv, out_hbm.at[wid])


@jax.jit
def _partials(rx, ry, rz, px, py, pz):
    mesh = plsc.VectorSubcoreMesh(
        core_axis_name="c", subcore_axis_name="s",
        num_cores=_NC, num_subcores=_NS)
    f = pl.kernel(
        _body,
        out_type=jax.ShapeDtypeStruct((_NW, _L), jnp.float32),
        mesh=mesh,
        scratch_types=[
            pltpu.VMEM((_N,), jnp.float32),
            pltpu.VMEM((_N,), jnp.float32),
            pltpu.VMEM((_N,), jnp.float32),
            pltpu.VMEM((_N,), jnp.float32),
            pltpu.VMEM((_N,), jnp.float32),
            pltpu.VMEM((_N,), jnp.float32),
            pltpu.VMEM((_N,), jnp.float32),
            pltpu.VMEM((_HCAP,), jnp.int32),
            pltpu.VMEM((_HCAP,), jnp.int32),
            pltpu.VMEM((_HCAP,), jnp.int32),
            pltpu.VMEM((_HCAP,), jnp.int32),
            pltpu.VMEM((_L,), jnp.float32),
        ],
        compiler_params=pltpu.CompilerParams(needs_layout_passes=False),
    )
    return f(rx, ry, rz, px, py, pz)


def kernel(points_ref, points):
    rx, ry, rz = (points_ref[:, :, i].reshape(-1) for i in range(3))
    px, py, pz = (points[:, :, i].reshape(-1) for i in range(3))
    partials = _partials(rx, ry, rz, px, py, pz)
    return jnp.sum(partials) / jnp.float32(_B * _N * _K)


# speedup vs baseline: 2.8747x; 2.8747x over previous
import functools

import numpy as np
import jax
import jax.numpy as jnp
from jax import lax
from jax.experimental import pallas as pl
from jax.experimental.pallas import tpu as pltpu
from jax.experimental.pallas import tpu_sc as plsc

_B = 4
_N = 4096
_K = 16
_L = 16
_NBLK = _N // _L
_NC = 2
_NS = 16
_NW = _NC * _NS
_WPB = _NW // _B
_ROWS = _N // _WPB
_BIG = np.float32(3.0e38)
_SAMPLE = 256
_HCAP = _N + _L


def _sqrt16(a):
    i = plsc.bitcast(a, jnp.int32)
    i = jnp.int32(0x5F3759DF) - (i >> 1)
    y = plsc.bitcast(i, jnp.float32)
    ah = a * jnp.float32(0.5)
    y = y * (jnp.float32(1.5) - ah * y * y)
    y = y * (jnp.float32(1.5) - ah * y * y)
    y = y * (jnp.float32(1.5) - ah * y * y)
    return jnp.where(a > 0.0, a * y, jnp.float32(0.0))


def _body(rx_hbm, ry_hbm, rz_hbm, px_hbm, py_hbm, pz_hbm, out_hbm,
          xs, ys, zs, pxs, pys, pzs, sqc, hitv, hitv2, hitv3, hitv4, accv):
    wid = lax.axis_index("s") * _NC + lax.axis_index("c")
    batch = wid // _WPB
    row0 = (wid % _WPB) * _ROWS

    boff = batch * _N
    pltpu.sync_copy(rx_hbm.at[pl.ds(boff, _N)], xs)
    pltpu.sync_copy(ry_hbm.at[pl.ds(boff, _N)], ys)
    pltpu.sync_copy(rz_hbm.at[pl.ds(boff, _N)], zs)
    pltpu.sync_copy(px_hbm.at[pl.ds(boff, _N)], pxs)
    pltpu.sync_copy(py_hbm.at[pl.ds(boff, _N)], pys)
    pltpu.sync_copy(pz_hbm.at[pl.ds(boff, _N)], pzs)

    iota = lax.iota(jnp.int32, _L)

    @plsc.parallel_loop(0, _NBLK, unroll=4)
    def _sq(c):
        base = c * _L
        xv = xs[pl.ds(base, _L)]
        yv = ys[pl.ds(base, _L)]
        zv = zs[pl.ds(base, _L)]
        sqc[pl.ds(base, _L)] = xv * xv + yv * yv + zv * zv

    def key_block(base, q):
        xv = xs[pl.ds(base, _L)]
        yv = ys[pl.ds(base, _L)]
        zv = zs[pl.ds(base, _L)]
        sc = sqc[pl.ds(base, _L)]
        t0 = q[0] * xv + q[1] * yv + q[2] * zv
        return sc - 2.0 * t0

    def merge(carry, d2, idxv):
        bk, bv = carry
        sk, sv = plsc.sort_key_val(d2, idxv, descending=True)
        take = bk <= sk
        mk = jnp.where(take, bk, sk)
        mv = jnp.where(take, bv, sv)
        nk, nv = plsc.sort_key_val(mk, mv)
        return nk, nv

    def sample_top16x4(qs, rvs):
        def p1_body(c, carry):
            base = c * _L
            xv = xs[pl.ds(base, _L)]
            yv = ys[pl.ds(base, _L)]
            zv = zs[pl.ds(base, _L)]
            sc = sqc[pl.ds(base, _L)]
            idxv = iota + base
            out = []
            for q, rv, ch in zip(qs, rvs, carry):
                v = sc - 2.0 * (q[0] * xv + q[1] * yv + q[2] * zv)
                v = jnp.where(idxv == rv, _BIG, v)
                out.append(merge(ch, v, idxv))
            return tuple(out)

        bk0 = jnp.full((_L,), _BIG, jnp.float32)
        bv0 = jnp.zeros((_L,), jnp.int32)
        init = tuple((bk0, bv0) for _ in range(4))
        return lax.fori_loop(0, _SAMPLE // _L, p1_body, init)

    def merge_hits_x4(chains, cnts, hrefs, rvs, qs):
        def p3_body(j, carry):
            base = j * _L
            out = []
            for (bk, bv), cnt, hv_ref, rv, q in zip(
                    carry, cnts, hrefs, rvs, qs):
                valid = iota + base < cnt
                hv = hv_ref[pl.ds(base, _L)]
                cx = plsc.load_gather(xs, [hv], mask=valid)
                cy = plsc.load_gather(ys, [hv], mask=valid)
                cz = plsc.load_gather(zs, [hv], mask=valid)
                scv = plsc.load_gather(sqc, [hv], mask=valid)
                hk = scv - 2.0 * (q[0] * cx + q[1] * cy + q[2] * cz)
                hk = jnp.where(valid, hk, _BIG)
                hk = jnp.where(hv == rv, _BIG, hk)
                out.append(merge((bk, bv), hk, hv))
            return tuple(out)

        cmax = jnp.maximum(jnp.maximum(cnts[0], cnts[1]),
                           jnp.maximum(cnts[2], cnts[3]))
        nit = (cmax + _L - 1) // _L
        return lax.fori_loop(0, nit, p3_body, chains)

    def edge_loss(rv, bk, bv):
        sqq = plsc.load_gather(sqc, [rv])
        dist_ref = _sqrt16(bk + sqq)
        qpx = plsc.load_gather(pxs, [rv])
        qpy = plsc.load_gather(pys, [rv])
        qpz = plsc.load_gather(pzs, [rv])
        nx = plsc.load_gather(pxs, [bv])
        ny = plsc.load_gather(pys, [bv])
        nz = plsc.load_gather(pzs, [bv])
        ddx = nx - qpx
        ddy = ny - qpy
        ddz = nz - qpz
        dist = _sqrt16(ddx * ddx + ddy * ddy + ddz * ddz)
        return jnp.abs(dist_ref - dist)

    def quad_body(p, acc_comp):
        acc, comp = acc_comp
        ra = row0 + 4 * p
        rva = jnp.full((_L,), ra, jnp.int32)
        rvs = (rva, rva + 1, rva + 2, rva + 3)
        qs = tuple(tuple(plsc.load_gather(s, [rv]) for s in (xs, ys, zs))
                   for rv in rvs)

        chains = sample_top16x4(qs, rvs)
        ts = tuple(jnp.max(ch[0]) for ch in chains)

        hrefs = (hitv, hitv2, hitv3, hitv4)
        zero4 = (jnp.int32(0),) * 4

        @plsc.parallel_loop(_SAMPLE // _L, _NBLK, unroll=8, carry=zero4)
        def p2_cnt(c, cnts):
            base = c * _L
            xv = xs[pl.ds(base, _L)]
            yv = ys[pl.ds(base, _L)]
            zv = zs[pl.ds(base, _L)]
            sc = sqc[pl.ds(base, _L)]
            idxv = iota + base
            out = []
            for q, t, hv_ref, cnt in zip(qs, ts, hrefs, cnts):
                v = sc - 2.0 * (q[0] * xv + q[1] * yv + q[2] * zv)
                hit = v < t
                plsc.store_compressed(hv_ref.at[pl.ds(cnt, _L)], idxv, mask=hit)
                out.append(cnt + plsc.all_reduce_population_count(hit)[0])
            return tuple(out)

        chains = merge_hits_x4(chains, p2_cnt, hrefs, rvs, qs)
        term = jnp.zeros((_L,), jnp.float32)
        for (bk, bv), rv in zip(chains, rvs):
            term = term + edge_loss(rv, bk, bv)

        y = term - comp
        t = acc + y
        comp = (t - acc) - y
        return t, comp

    zero = jnp.zeros((_L,), jnp.float32)
    acc, _ = lax.fori_loop(0, _ROWS // 4, quad_body, (zero, zero))
    accv[...] = acc
    pltpu.sync_copy(accv, out_hbm.at[wid])


@jax.jit
def _partials(rx, ry, rz, px, py, pz):
    mesh = plsc.VectorSubcoreMesh(
        core_axis_name="c", subcore_axis_name="s",
        num_cores=_NC, num_subcores=_NS)
    f = pl.kernel(
        _body,
        out_type=jax.ShapeDtypeStruct((_NW, _L), jnp.float32),
        mesh=mesh,
        scratch_types=[
            pltpu.VMEM((_N,), jnp.float32),
            pltpu.VMEM((_N,), jnp.float32),
            pltpu.VMEM((_N,), jnp.float32),
            pltpu.VMEM((_N,), jnp.float32),
            pltpu.VMEM((_N,), jnp.float32),
            pltpu.VMEM((_N,), jnp.float32),
            pltpu.VMEM((_N,), jnp.float32),
            pltpu.VMEM((_HCAP,), jnp.int32),
            pltpu.VMEM((_HCAP,), jnp.int32),
            pltpu.VMEM((_HCAP,), jnp.int32),
            pltpu.VMEM((_HCAP,), jnp.int32),
            pltpu.VMEM((_L,), jnp.float32),
        ],
        compiler_params=pltpu.CompilerParams(needs_layout_passes=False),
    )
    return f(rx, ry, rz, px, py, pz)


def kernel(points_ref, points):
    rx, ry, rz = (points_ref[:, :, i].reshape(-1) for i in range(3))
    px, py, pz = (points[:, :, i].reshape(-1) for i in range(3))
    partials = _partials(rx, ry, rz, px, py, pz)
    return jnp.sum(partials) / jnp.float32(_B * _N * _K)


# speedup vs baseline: 4.5421x; 1.5800x over previous
import functools

import numpy as np
import jax
import jax.numpy as jnp
from jax import lax
from jax.experimental import pallas as pl
from jax.experimental.pallas import tpu as pltpu
from jax.experimental.pallas import tpu_sc as plsc

_B = 4
_N = 4096
_K = 16
_L = 16
_NBLK = _N // _L
_NC = 2
_NS = 16
_NW = _NC * _NS
_WPB = _NW // _B
_ROWS = _N // _WPB
_BIG = np.float32(3.0e38)
_SAMPLE = 256
_HCAP = _N + _L


def _sqrt16(a):
    i = plsc.bitcast(a, jnp.int32)
    i = jnp.int32(0x5F3759DF) - (i >> 1)
    y = plsc.bitcast(i, jnp.float32)
    ah = a * jnp.float32(0.5)
    y = y * (jnp.float32(1.5) - ah * y * y)
    y = y * (jnp.float32(1.5) - ah * y * y)
    y = y * (jnp.float32(1.5) - ah * y * y)
    return jnp.where(a > 0.0, a * y, jnp.float32(0.0))


def _body(rx_hbm, ry_hbm, rz_hbm, px_hbm, py_hbm, pz_hbm, out_hbm,
          xs, ys, zs, pxs, pys, pzs, sqc, hitv, hitv2, hitv3, hitv4, accv):
    wid = lax.axis_index("s") * _NC + lax.axis_index("c")
    batch = wid // _WPB
    row0 = (wid % _WPB) * _ROWS

    boff = batch * _N
    pltpu.sync_copy(rx_hbm.at[pl.ds(boff, _N)], xs)
    pltpu.sync_copy(ry_hbm.at[pl.ds(boff, _N)], ys)
    pltpu.sync_copy(rz_hbm.at[pl.ds(boff, _N)], zs)
    pltpu.sync_copy(px_hbm.at[pl.ds(boff, _N)], pxs)
    pltpu.sync_copy(py_hbm.at[pl.ds(boff, _N)], pys)
    pltpu.sync_copy(pz_hbm.at[pl.ds(boff, _N)], pzs)

    iota = lax.iota(jnp.int32, _L)

    @plsc.parallel_loop(0, _NBLK, unroll=4)
    def _sq(c):
        base = c * _L
        xv = xs[pl.ds(base, _L)]
        yv = ys[pl.ds(base, _L)]
        zv = zs[pl.ds(base, _L)]
        sqc[pl.ds(base, _L)] = xv * xv + yv * yv + zv * zv

    def key_block(base, q):
        xv = xs[pl.ds(base, _L)]
        yv = ys[pl.ds(base, _L)]
        zv = zs[pl.ds(base, _L)]
        sc = sqc[pl.ds(base, _L)]
        t0 = q[0] * xv + q[1] * yv + q[2] * zv
        return sc - 2.0 * t0

    def merge(carry, d2, idxv):
        bk, bv = carry
        sk, sv = plsc.sort_key_val(d2, idxv, descending=True)
        take = bk <= sk
        mk = jnp.where(take, bk, sk)
        mv = jnp.where(take, bv, sv)
        nk, nv = plsc.sort_key_val(mk, mv)
        return nk, nv

    def sample_top16x4(qs, rvs):
        def p1_body(c, carry):
            base = c * _L
            xv = xs[pl.ds(base, _L)]
            yv = ys[pl.ds(base, _L)]
            zv = zs[pl.ds(base, _L)]
            sc = sqc[pl.ds(base, _L)]
            idxv = iota + base
            out = []
            for q, rv, ch in zip(qs, rvs, carry):
                v = sc - 2.0 * (q[0] * xv + q[1] * yv + q[2] * zv)
                v = jnp.where(idxv == rv, _BIG, v)
                out.append(merge(ch, v, idxv))
            return tuple(out)

        bk0 = jnp.full((_L,), _BIG, jnp.float32)
        bv0 = jnp.zeros((_L,), jnp.int32)
        init = tuple((bk0, bv0) for _ in range(4))
        return lax.fori_loop(0, _SAMPLE // _L, p1_body, init)

    def merge_hits_x4(chains, cnts, hrefs, rvs, qs):
        def p3_body(j, carry):
            base = j * _L
            out = []
            for (bk, bv), cnt, hv_ref, rv, q in zip(
                    carry, cnts, hrefs, rvs, qs):
                valid = iota + base < cnt
                hv = hv_ref[pl.ds(base, _L)]
                cx = plsc.load_gather(xs, [hv], mask=valid)
                cy = plsc.load_gather(ys, [hv], mask=valid)
                cz = plsc.load_gather(zs, [hv], mask=valid)
                scv = plsc.load_gather(sqc, [hv], mask=valid)
                hk = scv - 2.0 * (q[0] * cx + q[1] * cy + q[2] * cz)
                hk = jnp.where(valid, hk, _BIG)
                hk = jnp.where(hv == rv, _BIG, hk)
                out.append(merge((bk, bv), hk, hv))
            return tuple(out)

        cmax = jnp.maximum(jnp.maximum(cnts[0], cnts[1]),
                           jnp.maximum(cnts[2], cnts[3]))
        nit = (cmax + _L - 1) // _L
        return lax.fori_loop(0, nit, p3_body, chains)

    def edge_loss(rv, bk, bv):
        sqq = plsc.load_gather(sqc, [rv])
        dist_ref = _sqrt16(bk + sqq)
        qpx = plsc.load_gather(pxs, [rv])
        qpy = plsc.load_gather(pys, [rv])
        qpz = plsc.load_gather(pzs, [rv])
        nx = plsc.load_gather(pxs, [bv])
        ny = plsc.load_gather(pys, [bv])
        nz = plsc.load_gather(pzs, [bv])
        ddx = nx - qpx
        ddy = ny - qpy
        ddz = nz - qpz
        dist = _sqrt16(ddx * ddx + ddy * ddy + ddz * ddz)
        return jnp.abs(dist_ref - dist)

    def quad_body(p, acc_comp):
        acc, comp = acc_comp
        ra = row0 + 4 * p
        rva = jnp.full((_L,), ra, jnp.int32)
        rvs = (rva, rva + 1, rva + 2, rva + 3)
        qs = tuple(tuple(plsc.load_gather(s, [rv]) for s in (xs, ys, zs))
                   for rv in rvs)

        chains = sample_top16x4(qs, rvs)
        ts = tuple(jnp.max(ch[0]) for ch in chains)

        hrefs = (hitv, hitv2, hitv3, hitv4)
        zero4 = (jnp.int32(0),) * 4

        @plsc.parallel_loop(_SAMPLE // _L, _NBLK, unroll=4, carry=zero4)
        def p2_cnt(c, cnts):
            base = c * _L
            xv = xs[pl.ds(base, _L)]
            yv = ys[pl.ds(base, _L)]
            zv = zs[pl.ds(base, _L)]
            sc = sqc[pl.ds(base, _L)]
            idxv = iota + base
            out = []
            for q, t, hv_ref, cnt in zip(qs, ts, hrefs, cnts):
                v = sc - 2.0 * (q[0] * xv + q[1] * yv + q[2] * zv)
                hit = v < t
                plsc.store_compressed(hv_ref.at[pl.ds(cnt, _L)], idxv, mask=hit)
                out.append(cnt + plsc.all_reduce_population_count(hit)[0])
            return tuple(out)

        chains = merge_hits_x4(chains, p2_cnt, hrefs, rvs, qs)
        term = jnp.zeros((_L,), jnp.float32)
        for (bk, bv), rv in zip(chains, rvs):
            term = term + edge_loss(rv, bk, bv)

        y = term - comp
        t = acc + y
        comp = (t - acc) - y
        return t, comp

    zero = jnp.zeros((_L,), jnp.float32)
    acc, _ = lax.fori_loop(0, _ROWS // 4, quad_body, (zero, zero))
    accv[...] = acc
    pltpu.sync_copy(accv, out_hbm.at[wid])


@jax.jit
def _partials(rx, ry, rz, px, py, pz):
    mesh = plsc.VectorSubcoreMesh(
        core_axis_name="c", subcore_axis_name="s",
        num_cores=_NC, num_subcores=_NS)
    f = pl.kernel(
        _body,
        out_type=jax.ShapeDtypeStruct((_NW, _L), jnp.float32),
        mesh=mesh,
        scratch_types=[
            pltpu.VMEM((_N,), jnp.float32),
            pltpu.VMEM((_N,), jnp.float32),
            pltpu.VMEM((_N,), jnp.float32),
            pltpu.VMEM((_N,), jnp.float32),
            pltpu.VMEM((_N,), jnp.float32),
            pltpu.VMEM((_N,), jnp.float32),
            pltpu.VMEM((_N,), jnp.float32),
            pltpu.VMEM((_HCAP,), jnp.int32),
            pltpu.VMEM((_HCAP,), jnp.int32),
            pltpu.VMEM((_HCAP,), jnp.int32),
            pltpu.VMEM((_HCAP,), jnp.int32),
            pltpu.VMEM((_L,), jnp.float32),
        ],
        compiler_params=pltpu.CompilerParams(needs_layout_passes=False),
    )
    return f(rx, ry, rz, px, py, pz)


def kernel(points_ref, points):
    rx, ry, rz = (points_ref[:, :, i].reshape(-1) for i in range(3))
    px, py, pz = (points[:, :, i].reshape(-1) for i in range(3))
    partials = _partials(rx, ry, rz, px, py, pz)
    return jnp.sum(partials) / jnp.float32(_B * _N * _K)
